# Initial kernel scaffold; baseline (speedup 1.0000x reference)
import functools

import jax
import jax.numpy as jnp
from jax import lax
from jax.experimental import pallas as pl
from jax.experimental.pallas import tpu as pltpu
from jax.experimental.pallas import tpu_sc as plsc

N = 10000
E = 320000
D = 128
B = 128
NC, NS = 2, 16
NW = NC * NS
ROWS = (E + B - 1) // B
ROWS_PAD = ((ROWS + NW - 1) // NW) * NW
BPW = ROWS_PAD // NW
NPAD = ((N + 1 + NS - 1) // NS) * NS
RPT = NPAD // NS
DW = 16


def _sc_scatter(feat, src2d, dst2d, z128, z16, ones16):
  mesh = plsc.VectorSubcoreMesh(core_axis_name="c", subcore_axis_name="s")

  @functools.partial(
      pl.kernel,
      out_type=[
          jax.ShapeDtypeStruct((NC, NPAD, D), jnp.float32),
          jax.ShapeDtypeStruct((NC, NPAD, DW), jnp.float32),
      ],
      mesh=mesh,
      scratch_types=[
          pltpu.VMEM((BPW, B), jnp.int32),
          pltpu.VMEM((BPW, B), jnp.int32),
          pltpu.VMEM((B, D), jnp.float32),
          pltpu.VMEM((B, DW), jnp.float32),
          pltpu.VMEM_SHARED((NPAD, D), jnp.float32),
          pltpu.VMEM_SHARED((NPAD, DW), jnp.float32),
          pltpu.SemaphoreType.DMA,
      ],
  )
  def k(feat_hbm, src_hbm, dst_hbm, z128_hbm, z16_hbm, ones_hbm,
        psum_hbm, pdeg_hbm, src_v, dst_v, gbuf, ones_v, ssum, sdeg, sem):
    c = lax.axis_index("c")
    s = lax.axis_index("s")
    wid = c * NS + s
    row0 = wid * BPW
    srow = s * RPT

    pltpu.sync_copy(z128_hbm, ssum.at[pl.ds(srow, RPT)])
    pltpu.sync_copy(z16_hbm, sdeg.at[pl.ds(srow, RPT)])
    pltpu.sync_copy(src_hbm.at[pl.ds(row0, BPW)], src_v)
    pltpu.sync_copy(dst_hbm.at[pl.ds(row0, BPW)], dst_v)
    pltpu.sync_copy(ones_hbm, ones_v)
    plsc.subcore_barrier()

    def body(j, carry):
      pltpu.async_copy(feat_hbm.at[src_v.at[j]], gbuf, sem).wait()
      pltpu.sync_copy(gbuf, ssum.at[dst_v.at[j]], add=True)
      pltpu.sync_copy(ones_v, sdeg.at[dst_v.at[j]], add=True)
      return carry

    lax.fori_loop(0, BPW, body, 0)
    plsc.subcore_barrier()

    pltpu.sync_copy(ssum.at[pl.ds(srow, RPT)], psum_hbm.at[c, pl.ds(srow, RPT)])
    pltpu.sync_copy(sdeg.at[pl.ds(srow, RPT)], pdeg_hbm.at[c, pl.ds(srow, RPT)])

  return k(feat, src2d, dst2d, z128, z16, ones16)


def _combine_body(ps_ref, pd_ref, o_ref):
  ssum = ps_ref[0] + ps_ref[1]
  deg = pd_ref[0, :, 0:1] + pd_ref[1, :, 0:1]
  o_ref[...] = ssum / jnp.maximum(deg, 1.0)


_ROWS_BLK = 1000


def _combine(psum, pdeg):
  return pl.pallas_call(
      _combine_body,
      grid=(N // _ROWS_BLK,),
      in_specs=[
          pl.BlockSpec((NC, _ROWS_BLK, D), lambda i: (0, i, 0)),
          pl.BlockSpec((NC, _ROWS_BLK, DW), lambda i: (0, i, 0)),
      ],
      out_specs=pl.BlockSpec((_ROWS_BLK, D), lambda i: (i, 0)),
      out_shape=jax.ShapeDtypeStruct((N, D), jnp.float32),
  )(psum, pdeg)


@jax.jit
def kernel(feat, edge_index):
  src = edge_index[0].astype(jnp.int32)
  dst = edge_index[1].astype(jnp.int32)
  pad = ROWS_PAD * B - E
  src2d = jnp.concatenate([src, jnp.zeros((pad,), jnp.int32)]).reshape(ROWS_PAD, B)
  dst2d = jnp.concatenate([dst, jnp.full((pad,), N, jnp.int32)]).reshape(ROWS_PAD, B)
  z128 = jnp.zeros((RPT, D), jnp.float32)
  z16 = jnp.zeros((RPT, DW), jnp.float32)
  ones16 = jnp.ones((B, DW), jnp.float32)
  psum, pdeg = _sc_scatter(feat, src2d, dst2d, z128, z16, ones16)
  return _combine(psum, pdeg)


# speedup vs baseline: 3.7222x; 3.7222x over previous
import dataclasses
import functools

import jax
import jax.numpy as jnp
from jax import lax
from jax.experimental import pallas as pl
from jax.experimental.pallas import tpu as pltpu
from jax.experimental.pallas import tpu_sc as plsc

N = 10000
E = 320000
D = 128
B = 128
NC, NS = 2, 16
NW = NC * NS
ROWS = (E + B - 1) // B
BPW = (-(-ROWS // NW) + 7) // 8 * 8
ROWS_PAD = BPW * NW
NPAD = -(-(N + 1) // (NS * 8)) * NS * 8
RPT = NPAD // NS
RB = 8
CHUNKS = BPW // RB


def _sc_scatter(featx, src2d, dst2d, z128):
  mesh = plsc.VectorSubcoreMesh(
      core_axis_name="c", subcore_axis_name="s", num_cores=NC, num_subcores=NS)
  cp = pltpu.CompilerParams()
  if "needs_layout_passes" in pltpu.CompilerParams.__dataclass_fields__:
    cp = dataclasses.replace(cp, needs_layout_passes=False)

  @functools.partial(
      pl.kernel,
      compiler_params=cp,
      out_type=[
          jax.ShapeDtypeStruct((NC, NPAD, D), jnp.float32),
          jax.ShapeDtypeStruct((NC, NS, NPAD), jnp.float32),
      ],
      mesh=mesh,
      scratch_types=[
          pltpu.VMEM((RB, B), jnp.int32),
          pltpu.VMEM((RB, B), jnp.int32),
          pltpu.VMEM((B, D), jnp.float32),
          pltpu.VMEM((NPAD,), jnp.float32),
          pltpu.VMEM_SHARED((NPAD, D), jnp.float32),
          pltpu.SemaphoreType.DMA,
      ],
  )
  def k(feat_hbm, src_hbm, dst_hbm, z128_hbm,
        psum_hbm, pdeg_hbm, src_v, dst_v, gbuf, hist_v, ssum, sem):
    c = lax.axis_index("c")
    s = lax.axis_index("s")
    wid = c * NS + s
    row0 = wid * BPW
    srow = s * RPT
    lane = lax.iota(jnp.int32, 16)

    pltpu.sync_copy(z128_hbm, ssum.at[pl.ds(srow, RPT)])

    def zro(i, carry):
      hist_v[pl.ds(i * 16, 16)] = jnp.zeros((16,), jnp.float32)
      return carry

    lax.fori_loop(0, NPAD // 16, zro, 0)
    plsc.subcore_barrier()

    def chunk(ci, carry):
      pltpu.sync_copy(src_hbm.at[pl.ds(row0 + ci * RB, RB)], src_v)
      pltpu.sync_copy(dst_hbm.at[pl.ds(row0 + ci * RB, RB)], dst_v)
      for j in range(RB):
        pltpu.async_copy(feat_hbm.at[src_v.at[j]], gbuf, sem).wait()
        pltpu.sync_copy(gbuf, ssum.at[dst_v.at[j]], add=True)

      def hst(g, carry2):
        dvec = dst_v[g // (B // 16), pl.ds((g % (B // 16)) * 16, 16)]
        for l in range(16):
          d = dvec[l]
          base = jnp.bitwise_and(d, -16)
          off = d - base
          w = hist_v[pl.ds(base, 16)]
          hist_v[pl.ds(base, 16)] = w + (lane == off).astype(jnp.float32)
        return carry2

      lax.fori_loop(0, RB * B // 16, hst, 0)
      return carry

    lax.fori_loop(0, CHUNKS, chunk, 0)
    plsc.subcore_barrier()

    pltpu.sync_copy(ssum.at[pl.ds(srow, RPT)], psum_hbm.at[c, pl.ds(srow, RPT)])
    pltpu.sync_copy(hist_v, pdeg_hbm.at[c, s])

  return k(featx, src2d, dst2d, z128)


def _combine_body(ps_ref, pd_ref, o_ref):
  ssum = ps_ref[0] + ps_ref[1]
  deg = jnp.sum(pd_ref[...], axis=(0, 1))
  o_ref[...] = ssum / jnp.maximum(deg, 1.0)[:, None]


_NBLK = 128


def _combine(psum, pdeg):
  return pl.pallas_call(
      _combine_body,
      grid=(NPAD // _NBLK,),
      in_specs=[
          pl.BlockSpec((NC, _NBLK, D), lambda i: (0, i, 0)),
          pl.BlockSpec((NC, NS, _NBLK), lambda i: (0, 0, i)),
      ],
      out_specs=pl.BlockSpec((_NBLK, D), lambda i: (i, 0)),
      out_shape=jax.ShapeDtypeStruct((NPAD, D), jnp.float32),
  )(psum, pdeg)


@jax.jit
def kernel(feat, edge_index):
  src = edge_index[0].astype(jnp.int32)
  dst = edge_index[1].astype(jnp.int32)
  pad = ROWS_PAD * B - E
  featx = jnp.concatenate([feat, jnp.zeros((16, D), feat.dtype)])
  src2d = jnp.concatenate([src, jnp.full((pad,), N, jnp.int32)]).reshape(ROWS_PAD, B)
  dst2d = jnp.concatenate([dst, jnp.full((pad,), N, jnp.int32)]).reshape(ROWS_PAD, B)
  z128 = jnp.zeros((RPT, D), jnp.float32)
  psum, pdeg = _sc_scatter(featx, src2d, dst2d, z128)
  return _combine(psum, pdeg)[:N]


# speedup vs baseline: 3.8142x; 1.0247x over previous
import dataclasses
import functools

import jax
import jax.numpy as jnp
from jax import lax
from jax.experimental import pallas as pl
from jax.experimental.pallas import tpu as pltpu
from jax.experimental.pallas import tpu_sc as plsc

N = 10000
E = 320000
D = 128
B = 128
NC, NS = 2, 16
NW = NC * NS
ROWS = (E + B - 1) // B
BPW = (-(-ROWS // NW) + 7) // 8 * 8
ROWS_PAD = BPW * NW
NPAD = -(-(N + 1) // (NS * 8)) * NS * 8
RPT = NPAD // NS
HW = NPAD // 2
RB = 8
CHUNKS = BPW // RB


def _sc_scatter(featx, src2d, dst2d, z128):
  mesh = plsc.VectorSubcoreMesh(
      core_axis_name="c", subcore_axis_name="s", num_cores=NC, num_subcores=NS)
  cp = pltpu.CompilerParams()
  if "needs_layout_passes" in pltpu.CompilerParams.__dataclass_fields__:
    cp = dataclasses.replace(cp, needs_layout_passes=False)

  @functools.partial(
      pl.kernel,
      compiler_params=cp,
      out_type=[
          jax.ShapeDtypeStruct((NC, NPAD, D), jnp.float32),
          jax.ShapeDtypeStruct((NC, NS, HW), jnp.int32),
      ],
      mesh=mesh,
      scratch_types=[
          pltpu.VMEM((RB, B), jnp.int32),
          pltpu.VMEM((RB, B), jnp.int32),
          pltpu.VMEM((B, D), jnp.float32),
          pltpu.VMEM((B, D), jnp.float32),
          pltpu.VMEM((HW,), jnp.int32),
          pltpu.VMEM_SHARED((NPAD, D), jnp.float32),
          pltpu.SemaphoreType.DMA,
          pltpu.SemaphoreType.DMA,
      ],
  )
  def k(feat_hbm, src_hbm, dst_hbm, z128_hbm,
        psum_hbm, pdeg_hbm, src_v, dst_v, gb0, gb1, hist_v, ssum,
        gsem, scsem):
    c = lax.axis_index("c")
    s = lax.axis_index("s")
    wid = c * NS + s
    row0 = wid * BPW
    srow = s * RPT
    lane = lax.iota(jnp.int32, 16)
    gb = (gb0, gb1)

    pltpu.sync_copy(z128_hbm, ssum.at[pl.ds(srow, RPT)])

    def zro(i, carry):
      hist_v[pl.ds(i * 16, 16)] = jnp.zeros((16,), jnp.int32)
      return carry

    lax.fori_loop(0, HW // 16, zro, 0)
    plsc.subcore_barrier()

    def hist_batch(j):
      def hst(g, carry2):
        dvec = dst_v[j, pl.ds(g * 16, 16)]
        for l in range(16):
          d = dvec[l]
          wi = lax.shift_right_logical(d, 1)
          base = jnp.bitwise_and(wi, -16)
          off = wi - base
          addv = lax.shift_left(1, jnp.bitwise_and(d, 1) * 16)
          w = hist_v[pl.ds(base, 16)]
          hist_v[pl.ds(base, 16)] = w + jnp.where(lane == off, addv, 0)
        return carry2

      lax.fori_loop(0, B // 16, hst, 0)

    def gather(j, buf):
      return pltpu.async_copy(feat_hbm.at[src_v.at[j]], buf, gsem)

    def scatter(j, buf):
      return pltpu.async_copy(buf, ssum.at[dst_v.at[j]], scsem, add=True)

    def wait_sc():
      pltpu.make_async_copy(gb1, ssum.at[dst_v.at[RB - 1]], scsem).wait()

    def chunk_body(ci, first):
      pltpu.sync_copy(src_hbm.at[pl.ds(row0 + ci * RB, RB)], src_v)
      g0 = gather(0, gb0)
      if not first:
        wait_sc()
      pltpu.sync_copy(dst_hbm.at[pl.ds(row0 + ci * RB, RB)], dst_v)
      g0.wait()
      gnext = gather(1, gb1)
      sc = scatter(0, gb0)
      hist_batch(0)
      for j in range(1, RB):
        gnext.wait()
        sc.wait()
        if j < RB - 1:
          gnext = gather(j + 1, gb[(j + 1) % 2])
        sc = scatter(j, gb[j % 2])
        hist_batch(j)

    chunk_body(0, True)

    def chunk(ci, carry):
      chunk_body(ci, False)
      return carry

    lax.fori_loop(1, CHUNKS, chunk, 0)
    wait_sc()
    plsc.subcore_barrier()

    pltpu.sync_copy(ssum.at[pl.ds(srow, RPT)], psum_hbm.at[c, pl.ds(srow, RPT)])
    pltpu.sync_copy(hist_v, pdeg_hbm.at[c, s])

  return k(featx, src2d, dst2d, z128)


def _combine_body(ps_ref, pd_ref, o_ref):
  ssum = ps_ref[0] + ps_ref[1]
  deg = jnp.sum(pd_ref[...].astype(jnp.float32), axis=(0, 1))
  o_ref[...] = ssum / jnp.maximum(deg, 1.0)[:, None]


_NBLK = 128


def _combine(psum, deg16):
  return pl.pallas_call(
      _combine_body,
      grid=(NPAD // _NBLK,),
      in_specs=[
          pl.BlockSpec((NC, _NBLK, D), lambda i: (0, i, 0)),
          pl.BlockSpec((NC, NS, _NBLK), lambda i: (0, 0, i)),
      ],
      out_specs=pl.BlockSpec((_NBLK, D), lambda i: (i, 0)),
      out_shape=jax.ShapeDtypeStruct((NPAD, D), jnp.float32),
  )(psum, deg16)


@jax.jit
def kernel(feat, edge_index):
  src = edge_index[0].astype(jnp.int32)
  dst = edge_index[1].astype(jnp.int32)
  pad = ROWS_PAD * B - E
  featx = jnp.concatenate([feat, jnp.zeros((16, D), feat.dtype)])
  src2d = jnp.concatenate([src, jnp.full((pad,), N, jnp.int32)]).reshape(ROWS_PAD, B)
  dst2d = jnp.concatenate([dst, jnp.full((pad,), N, jnp.int32)]).reshape(ROWS_PAD, B)
  z128 = jnp.zeros((RPT, D), jnp.float32)
  psum, pdeg = _sc_scatter(featx, src2d, dst2d, z128)
  deg16 = lax.bitcast_convert_type(pdeg, jnp.int16).reshape(NC, NS, NPAD)
  return _combine(psum, deg16)[:N]


# speedup vs baseline: 8.4861x; 2.2249x over previous
import dataclasses
import functools

import jax
import jax.numpy as jnp
from jax import lax
from jax.experimental import pallas as pl
from jax.experimental.pallas import tpu as pltpu
from jax.experimental.pallas import tpu_sc as plsc

N = 10000
E = 320000
D = 128
B = 128
NC, NS = 2, 16
NW = NC * NS
ROWS = (E + B - 1) // B
BPW = (-(-ROWS // NW) + 7) // 8 * 8
ROWS_PAD = BPW * NW
NPAD = -(-(N + 1) // (NS * 8)) * NS * 8
RPT = NPAD // NS
HW = NPAD // 2
RB = 8
CHUNKS = BPW // RB


def _sc_scatter(featx, src2d, dst2d, z128):
  mesh = plsc.VectorSubcoreMesh(
      core_axis_name="c", subcore_axis_name="s", num_cores=NC, num_subcores=NS)
  cp = pltpu.CompilerParams()
  if "needs_layout_passes" in pltpu.CompilerParams.__dataclass_fields__:
    cp = dataclasses.replace(cp, needs_layout_passes=False)

  @functools.partial(
      pl.kernel,
      compiler_params=cp,
      out_type=[
          jax.ShapeDtypeStruct((NC, NPAD, D), jnp.float32),
          jax.ShapeDtypeStruct((NC, NS, HW), jnp.int32),
      ],
      mesh=mesh,
      scratch_types=[
          pltpu.VMEM((RB, B), jnp.int32),
          pltpu.VMEM((RB, B), jnp.int32),
          pltpu.VMEM((B, D), jnp.float32),
          pltpu.VMEM((B, D), jnp.float32),
          pltpu.VMEM((HW,), jnp.int32),
          pltpu.VMEM_SHARED((NPAD, D), jnp.float32),
          pltpu.SemaphoreType.DMA,
          pltpu.SemaphoreType.DMA,
      ],
  )
  def k(feat_hbm, src_hbm, dst_hbm, z128_hbm,
        psum_hbm, pdeg_hbm, src_v, dst_v, gb0, gb1, hist_v, ssum,
        gsem, scsem):
    c = lax.axis_index("c")
    s = lax.axis_index("s")
    wid = c * NS + s
    row0 = wid * BPW
    srow = s * RPT
    lane = lax.iota(jnp.int32, 16)
    gb = (gb0, gb1)

    pltpu.sync_copy(z128_hbm, ssum.at[pl.ds(srow, RPT)])

    def zro(i, carry):
      hist_v[pl.ds(i * 16, 16)] = jnp.zeros((16,), jnp.int32)
      return carry

    lax.fori_loop(0, HW // 16, zro, 0)
    plsc.subcore_barrier()

    def hist_batch(j):
      def hst(g, carry2):
        dvec = dst_v[j, pl.ds(g * 16, 16)]
        for l in range(16):
          d = dvec[l]
          wi = lax.shift_right_logical(d, 1)
          base = jnp.bitwise_and(wi, -16)
          off = wi - base
          addv = lax.shift_left(1, jnp.bitwise_and(d, 1) * 16)
          w = hist_v[pl.ds(base, 16)]
          hist_v[pl.ds(base, 16)] = w + jnp.where(lane == off, addv, 0)
        return carry2

      lax.fori_loop(0, B // 16, hst, 0)

    def gather(j, buf):
      return pltpu.async_copy(feat_hbm.at[src_v.at[j]], buf, gsem)

    def scatter(j, buf):
      return pltpu.async_copy(buf, ssum.at[dst_v.at[j]], scsem, add=True)

    def wait_sc():
      pltpu.make_async_copy(gb1, ssum.at[dst_v.at[RB - 1]], scsem).wait()

    def chunk_body(ci, first):
      pltpu.sync_copy(src_hbm.at[pl.ds(row0 + ci * RB, RB)], src_v)
      g0 = gather(0, gb0)
      if not first:
        wait_sc()
      pltpu.sync_copy(dst_hbm.at[pl.ds(row0 + ci * RB, RB)], dst_v)
      g0.wait()
      gnext = gather(1, gb1)
      sc = scatter(0, gb0)
      hist_batch(0)
      for j in range(1, RB):
        gnext.wait()
        sc.wait()
        if j < RB - 1:
          gnext = gather(j + 1, gb[(j + 1) % 2])
        sc = scatter(j, gb[j % 2])
        hist_batch(j)

    chunk_body(0, True)

    def chunk(ci, carry):
      chunk_body(ci, False)
      return carry

    lax.fori_loop(1, CHUNKS, chunk, 0)
    wait_sc()
    plsc.subcore_barrier()

    pltpu.sync_copy(ssum.at[pl.ds(srow, RPT)], psum_hbm.at[c, pl.ds(srow, RPT)])
    pltpu.sync_copy(hist_v, pdeg_hbm.at[c, s])

  return k(featx, src2d, dst2d, z128)


def _combine_body(ps_ref, pd_ref, o_ref):
  ssum = ps_ref[0] + ps_ref[1]
  deg = jnp.sum(pd_ref[...].astype(jnp.float32), axis=(0, 1))
  o_ref[...] = ssum / jnp.maximum(deg, 1.0)[:, None]


_NBLK = 128


def _combine(psum, deg16):
  return pl.pallas_call(
      _combine_body,
      grid=(NPAD // _NBLK,),
      in_specs=[
          pl.BlockSpec((NC, _NBLK, D), lambda i: (0, i, 0)),
          pl.BlockSpec((NC, NS, _NBLK), lambda i: (0, 0, i)),
      ],
      out_specs=pl.BlockSpec((_NBLK, D), lambda i: (i, 0)),
      out_shape=jax.ShapeDtypeStruct((NPAD, D), jnp.float32),
  )(psum, deg16)


@jax.jit
def kernel(feat, edge_index):
  src = edge_index[0].astype(jnp.int32)
  dst = edge_index[1].astype(jnp.int32)
  pad = ROWS_PAD * B - E
  featx = jnp.concatenate([feat, jnp.zeros((16, D), feat.dtype)])
  pad_src = N + jnp.arange(pad, dtype=jnp.int32) % 16
  pad_dst = N + jnp.arange(pad, dtype=jnp.int32) % (NPAD - N)
  src2d = jnp.concatenate([src, pad_src]).reshape(ROWS_PAD, B)
  dst2d = jnp.concatenate([dst, pad_dst]).reshape(ROWS_PAD, B)
  z128 = jnp.zeros((RPT, D), jnp.float32)
  psum, pdeg = _sc_scatter(featx, src2d, dst2d, z128)
  deg16 = lax.bitcast_convert_type(pdeg, jnp.int16).reshape(NC, NS, NPAD)
  return _combine(psum, deg16)[:N]


# speedup vs baseline: 9.8613x; 1.1620x over previous
import dataclasses
import functools

import jax
import jax.numpy as jnp
from jax import lax
from jax.experimental import pallas as pl
from jax.experimental.pallas import tpu as pltpu
from jax.experimental.pallas import tpu_sc as plsc

N = 10000
E = 320000
D = 128
B = 128
NC, NS = 2, 16
NW = NC * NS
ROWS = (E + B - 1) // B
BPW = (-(-ROWS // NW) + 7) // 8 * 8
ROWS_PAD = BPW * NW
NPAD = -(-(N + 1) // (NS * 8)) * NS * 8
RPT = NPAD // NS
HW = NPAD // 2
RB = 8
CHUNKS = BPW // RB


def _sc_scatter(featx, src2d, dst2d, z128):
  mesh = plsc.VectorSubcoreMesh(
      core_axis_name="c", subcore_axis_name="s", num_cores=NC, num_subcores=NS)
  cp = pltpu.CompilerParams()
  if "needs_layout_passes" in pltpu.CompilerParams.__dataclass_fields__:
    cp = dataclasses.replace(cp, needs_layout_passes=False)

  @functools.partial(
      pl.kernel,
      compiler_params=cp,
      out_type=[
          jax.ShapeDtypeStruct((NC, NPAD, D), jnp.float32),
          jax.ShapeDtypeStruct((NC, NS, HW), jnp.int32),
      ],
      mesh=mesh,
      scratch_types=[
          pltpu.VMEM((RB, B), jnp.int32),
          pltpu.VMEM((RB, B), jnp.int32),
          pltpu.VMEM((B, D), jnp.float32),
          pltpu.VMEM((B, D), jnp.float32),
          pltpu.VMEM((HW,), jnp.int32),
          pltpu.VMEM_SHARED((NPAD, D), jnp.float32),
          pltpu.SemaphoreType.DMA,
          pltpu.SemaphoreType.DMA,
      ],
  )
  def k(feat_hbm, src_hbm, dst_hbm, z128_hbm,
        psum_hbm, pdeg_hbm, src_v, dst_v, gb0, gb1, hist_v, ssum,
        gsem, scsem):
    c = lax.axis_index("c")
    s = lax.axis_index("s")
    wid = c * NS + s
    row0 = wid * BPW
    srow = s * RPT
    lane = lax.iota(jnp.int32, 16)
    gb = (gb0, gb1)

    pltpu.sync_copy(z128_hbm, ssum.at[pl.ds(srow, RPT)])

    def zro(i, carry):
      hist_v[pl.ds(i * 16, 16)] = jnp.zeros((16,), jnp.int32)
      return carry

    lax.fori_loop(0, HW // 16, zro, 0)
    plsc.subcore_barrier()

    def hist_batch(j):
      def hst(g, carry2):
        dvec = dst_v[j, pl.ds(g * 16, 16)]
        for l in range(16):
          d = dvec[l]
          wi = lax.shift_right_logical(d, 1)
          base = jnp.bitwise_and(wi, -16)
          off = wi - base
          addv = lax.shift_left(1, jnp.bitwise_and(d, 1) * 16)
          w = hist_v[pl.ds(base, 16)]
          hist_v[pl.ds(base, 16)] = w + jnp.where(lane == off, addv, 0)
        return carry2

      lax.fori_loop(0, B // 16, hst, 0)

    def gather(j, buf):
      return pltpu.async_copy(feat_hbm.at[src_v.at[j]], buf, gsem)

    def scatter(j, buf):
      return pltpu.async_copy(buf, ssum.at[dst_v.at[j]], scsem, add=True)

    def wait_sc():
      pltpu.make_async_copy(gb1, ssum.at[dst_v.at[RB - 1]], scsem).wait()

    def chunk_body(ci, first):
      pltpu.sync_copy(src_hbm.at[pl.ds(row0 + ci * RB, RB)], src_v)
      g0 = gather(0, gb0)
      if not first:
        wait_sc()
      pltpu.sync_copy(dst_hbm.at[pl.ds(row0 + ci * RB, RB)], dst_v)
      g0.wait()
      gnext = gather(1, gb1)
      sc = scatter(0, gb0)
      hist_batch(0)
      for j in range(1, RB):
        gnext.wait()
        sc.wait()
        if j < RB - 1:
          gnext = gather(j + 1, gb[(j + 1) % 2])
        sc = scatter(j, gb[j % 2])
        hist_batch(j)

    chunk_body(0, True)

    def chunk(ci, carry):
      chunk_body(ci, False)
      return carry

    lax.fori_loop(1, CHUNKS, chunk, 0)
    wait_sc()
    plsc.subcore_barrier()

    pltpu.sync_copy(ssum.at[pl.ds(srow, RPT)], psum_hbm.at[c, pl.ds(srow, RPT)])
    pltpu.sync_copy(hist_v, pdeg_hbm.at[c, s])

  return k(featx, src2d, dst2d, z128)


def _combine_body(ps_ref, pd_ref, o_ref):
  ssum = ps_ref[0] + ps_ref[1]
  deg = jnp.sum(pd_ref[...].astype(jnp.float32), axis=(0, 1))
  rdeg = 1.0 / jnp.maximum(deg, 1.0)
  o_ref[...] = ssum * rdeg[:, None]


def _combine(psum, deg16):
  return pl.pallas_call(
      _combine_body,
      out_shape=jax.ShapeDtypeStruct((NPAD, D), jnp.float32),
  )(psum, deg16)


@jax.jit
def kernel(feat, edge_index):
  src = edge_index[0].astype(jnp.int32)
  dst = edge_index[1].astype(jnp.int32)
  pad = ROWS_PAD * B - E
  featx = jnp.concatenate([feat, jnp.zeros((16, D), feat.dtype)])
  pad_src = N + jnp.arange(pad, dtype=jnp.int32) % 16
  pad_dst = N + jnp.arange(pad, dtype=jnp.int32) % (NPAD - N)
  src2d = jnp.concatenate([src, pad_src]).reshape(ROWS_PAD, B)
  dst2d = jnp.concatenate([dst, pad_dst]).reshape(ROWS_PAD, B)
  z128 = jnp.zeros((RPT, D), jnp.float32)
  psum, pdeg = _sc_scatter(featx, src2d, dst2d, z128)
  deg16 = lax.bitcast_convert_type(pdeg, jnp.int16).reshape(NC, NS, NPAD)
  return _combine(psum, deg16)[:N]
